# baseline (speedup 1.0000x reference)
import functools
import math

import jax
import jax.numpy as jnp
from jax import lax
from jax.experimental import pallas as pl
from jax.experimental.pallas import tpu as pltpu
from jax.experimental.pallas import tpu_sc as plsc

CLASS_TYPE_STATIC = 2
CLASS_TYPE_NUM = 7
DIM = 10
HIDDEN = 64
POS_DIM = 4 + CLASS_TYPE_NUM

_SQRT_HALF = 1.0 / math.sqrt(2.0)
MB = 8


def _fused_kernel(xt_ref, w1t_ref, b1_ref, w2t_ref, b2_ref,
                  out_ref, mask_ref):
    w1t = w1t_ref[...]
    w2t = w2t_ref[...]
    b1 = b1_ref[...]
    b2 = b2_ref[...]
    for b in range(MB):
        xt = xt_ref[:, b, :]
        nonzero = jnp.sum((xt != 0.0).astype(jnp.float32), axis=0,
                          keepdims=True)
        mask_ref[b, :] = (nonzero == 0.0)[0]
        h = jnp.dot(w1t, xt, preferred_element_type=jnp.float32) + b1
        h = 0.5 * h * (1.0 + jax.lax.erf(h * _SQRT_HALF))
        proj = jnp.dot(w2t, h, preferred_element_type=jnp.float32) + b2
        validf = jnp.minimum(nonzero, 1.0)
        out_ref[b] = proj * validf


def _make_pos_kernel(B, P):
    info = plsc.get_sparse_core_info()
    NC, NS, L = info.num_cores, info.num_subcores, info.num_lanes
    NW = NC * NS
    rows = 8
    ncol = P // 2
    mesh = plsc.VectorSubcoreMesh(core_axis_name="c", subcore_axis_name="s")

    @functools.partial(
        pl.kernel, mesh=mesh,
        out_type=jax.ShapeDtypeStruct((POS_DIM, B, P), jnp.float32),
        scratch_types=[pltpu.VMEM((rows, ncol), jnp.float32)],
    )
    def pos_kernel(x_hbm, pos_hbm, const_v):
        wid = lax.axis_index("s") * NC + lax.axis_index("c")
        rbase = (wid // 2) * rows
        cbase = (wid % 2) * ncol

        for p in range(4):
            pltpu.sync_copy(x_hbm.at[p, pl.ds(rbase, rows), pl.ds(cbase, ncol)],
                            pos_hbm.at[p, pl.ds(rbase, rows), pl.ds(cbase, ncol)])

        def fill(val):
            def body(i, _):
                r = i // (ncol // L)
                c = (i % (ncol // L)) * L
                const_v[r, pl.ds(c, L)] = jnp.full((L,), val, jnp.float32)
                return 0
            lax.fori_loop(0, rows * (ncol // L), body, 0)

        fill(0.0)
        for p in range(CLASS_TYPE_NUM):
            if p != CLASS_TYPE_STATIC:
                pltpu.sync_copy(
                    const_v,
                    pos_hbm.at[4 + p, pl.ds(rbase, rows), pl.ds(cbase, ncol)])
        fill(1.0)
        pltpu.sync_copy(
            const_v,
            pos_hbm.at[4 + CLASS_TYPE_STATIC, pl.ds(rbase, rows),
                       pl.ds(cbase, ncol)])

    return pos_kernel


def kernel(x, W1, b1, W2, b2):
    B, P, D = x.shape
    xt = jnp.transpose(x, (2, 0, 1))

    grid = (B // MB,)
    out_t, mask = pl.pallas_call(
        _fused_kernel,
        grid=grid,
        in_specs=[
            pl.BlockSpec((D, MB, P), lambda i: (0, i, 0)),
            pl.BlockSpec((HIDDEN, D), lambda i: (0, 0)),
            pl.BlockSpec((HIDDEN, 1), lambda i: (0, 0)),
            pl.BlockSpec((HIDDEN, HIDDEN), lambda i: (0, 0)),
            pl.BlockSpec((HIDDEN, 1), lambda i: (0, 0)),
        ],
        out_specs=[
            pl.BlockSpec((MB, HIDDEN, P), lambda i: (i, 0, 0)),
            pl.BlockSpec((MB, P), lambda i: (i, 0)),
        ],
        out_shape=[
            jax.ShapeDtypeStruct((B, HIDDEN, P), jnp.float32),
            jax.ShapeDtypeStruct((B, P), jnp.bool_),
        ],
    )(xt, W1.T, b1.reshape(HIDDEN, 1), W2.T, b2.reshape(HIDDEN, 1))

    pos_t = _make_pos_kernel(B, P)(xt)

    out = jnp.transpose(out_t, (0, 2, 1))
    pos = jnp.transpose(pos_t, (1, 2, 0))
    return (out, mask, pos)


# speedup vs baseline: 2.8265x; 2.8265x over previous
import math

import jax
import jax.numpy as jnp
from jax.experimental import pallas as pl

CLASS_TYPE_STATIC = 2
CLASS_TYPE_NUM = 7
DIM = 10
HIDDEN = 64
POS_DIM = 4 + CLASS_TYPE_NUM

_SQRT_HALF = 1.0 / math.sqrt(2.0)
MB = 8


def _fused_kernel(xt_ref, w1t_ref, b1_ref, w2t_ref, b2_ref,
                  out_ref, mask_ref, pos_ref):
    w1t = w1t_ref[...]
    w2t = w2t_ref[...]
    b1 = b1_ref[...]
    b2 = b2_ref[...]
    for b in range(MB):
        xt = xt_ref[:, b, :]
        p = xt.shape[1]

        nonzero = jnp.sum((xt != 0.0).astype(jnp.float32), axis=0,
                          keepdims=True)
        mask_ref[b, :] = (nonzero == 0.0)[0]

        zeros_p = jnp.zeros((1, p), dtype=jnp.float32)
        ones_p = jnp.ones((1, p), dtype=jnp.float32)
        pre = [zeros_p] * CLASS_TYPE_STATIC
        post = [zeros_p] * (CLASS_TYPE_NUM - CLASS_TYPE_STATIC - 1)
        pos_ref[:, b, :] = jnp.concatenate([xt[:4]] + pre + [ones_p] + post,
                                           axis=0)

        h = jnp.dot(w1t, xt, preferred_element_type=jnp.float32) + b1
        h = 0.5 * h * (1.0 + jax.lax.erf(h * _SQRT_HALF))
        proj = jnp.dot(w2t, h, preferred_element_type=jnp.float32) + b2
        validf = jnp.minimum(nonzero, 1.0)
        out_ref[b] = proj * validf


def kernel(x, W1, b1, W2, b2):
    B, P, D = x.shape
    xt = jnp.transpose(x, (2, 0, 1))

    grid = (B // MB,)
    out_t, mask, pos_t = pl.pallas_call(
        _fused_kernel,
        grid=grid,
        in_specs=[
            pl.BlockSpec((D, MB, P), lambda i: (0, i, 0)),
            pl.BlockSpec((HIDDEN, D), lambda i: (0, 0)),
            pl.BlockSpec((HIDDEN, 1), lambda i: (0, 0)),
            pl.BlockSpec((HIDDEN, HIDDEN), lambda i: (0, 0)),
            pl.BlockSpec((HIDDEN, 1), lambda i: (0, 0)),
        ],
        out_specs=[
            pl.BlockSpec((MB, HIDDEN, P), lambda i: (i, 0, 0)),
            pl.BlockSpec((MB, P), lambda i: (i, 0)),
            pl.BlockSpec((POS_DIM, MB, P), lambda i: (0, i, 0)),
        ],
        out_shape=[
            jax.ShapeDtypeStruct((B, HIDDEN, P), jnp.float32),
            jax.ShapeDtypeStruct((B, P), jnp.bool_),
            jax.ShapeDtypeStruct((POS_DIM, B, P), jnp.float32),
        ],
    )(xt, W1.T, b1.reshape(HIDDEN, 1), W2.T, b2.reshape(HIDDEN, 1))

    out = jnp.transpose(out_t, (0, 2, 1))
    pos = jnp.transpose(pos_t, (1, 2, 0))
    return (out, mask, pos)
